# baseline (speedup 1.0000x reference)
import functools

import jax
import jax.numpy as jnp
from jax.experimental import pallas as pl

_T = 16384
_DIM = 4096
_E = 64
_BT = 256


def _gate_block(x_ref, w_ref, out_ref, idx_ref):
    scores = jax.lax.dot_general(
        x_ref[...], w_ref[...],
        dimension_numbers=(((1,), (1,)), ((), ())),
        preferred_element_type=jnp.float32,
    )
    m = jnp.max(scores, axis=-1, keepdims=True)
    e = jnp.exp(scores - m)
    s = jnp.sum(e, axis=-1, keepdims=True)
    p = 1.0 / s
    idx = jnp.argmax(scores, axis=-1)[:, None]
    out_ref[...] = (1.0 - p) + p
    idx_ref[...] = idx.astype(jnp.int32)


@jax.jit
def kernel(x, W):
    grid = (_T // _BT,)
    out, idx = pl.pallas_call(
        _gate_block,
        grid=grid,
        in_specs=[
            pl.BlockSpec((_BT, _DIM), lambda i: (i, 0)),
            pl.BlockSpec((_E, _DIM), lambda i: (0, 0)),
        ],
        out_specs=[
            pl.BlockSpec((_BT, 1), lambda i: (i, 0)),
            pl.BlockSpec((_BT, 1), lambda i: (i, 0)),
        ],
        out_shape=[
            jax.ShapeDtypeStruct((_T, 1), jnp.float32),
            jax.ShapeDtypeStruct((_T, 1), jnp.int32),
        ],
    )(x, W)
    return (out, idx)


# speedup vs baseline: 1.2159x; 1.2159x over previous
import functools

import jax
import jax.numpy as jnp
from jax.experimental import pallas as pl

_T = 16384
_DIM = 4096
_E = 64
_BT = 1024


def _gate_block(x_ref, w_ref, out_ref, idx_ref):
    scores = jax.lax.dot_general(
        x_ref[...], w_ref[...],
        dimension_numbers=(((1,), (1,)), ((), ())),
        preferred_element_type=jnp.float32,
    )
    m = jnp.max(scores, axis=-1, keepdims=True)
    e = jnp.exp(scores - m)
    s = jnp.sum(e, axis=-1, keepdims=True)
    p = 1.0 / s
    idx = jnp.argmax(scores, axis=-1)[:, None]
    out_ref[...] = (1.0 - p) + p
    idx_ref[...] = idx.astype(jnp.int32)


@jax.jit
def kernel(x, W):
    grid = (_T // _BT,)
    out, idx = pl.pallas_call(
        _gate_block,
        grid=grid,
        in_specs=[
            pl.BlockSpec((_BT, _DIM), lambda i: (i, 0)),
            pl.BlockSpec((_E, _DIM), lambda i: (0, 0)),
        ],
        out_specs=[
            pl.BlockSpec((_BT, 1), lambda i: (i, 0)),
            pl.BlockSpec((_BT, 1), lambda i: (i, 0)),
        ],
        out_shape=[
            jax.ShapeDtypeStruct((_T, 1), jnp.float32),
            jax.ShapeDtypeStruct((_T, 1), jnp.int32),
        ],
    )(x, W)
    return (out, idx)
